# baseline (speedup 1.0000x reference)
import functools

import jax
import jax.numpy as jnp
from jax import lax
from jax.experimental import pallas as pl
from jax.experimental.pallas import tpu as pltpu
from jax.experimental.pallas import tpu_sc as plsc

OUT = 1024
NF = 65536
B = 1024

NC = 2
NS = 16
L = 16
NW = NC * NS
RPW = B // NW
CH = 512
NCHUNK = NF // CH
XP = CH + 1
AP = 2 * OUT + 1

_mesh = plsc.VectorSubcoreMesh(
    core_axis_name="c", subcore_axis_name="s", num_cores=NC, num_subcores=NS
)


@functools.partial(
    pl.kernel,
    out_type=jax.ShapeDtypeStruct((B, OUT), jnp.float32),
    mesh=_mesh,
    compiler_params=pltpu.CompilerParams(use_tc_tiling_on_sc=False,
                                          needs_layout_passes=False),
    scratch_types=[
        pltpu.VMEM((RPW, XP), jnp.float32),
        pltpu.VMEM((RPW, XP), jnp.float32),
        pltpu.VMEM((CH,), jnp.int32),
        pltpu.VMEM((CH,), jnp.int32),
        pltpu.VMEM((CH,), jnp.int32),
        pltpu.VMEM((CH,), jnp.int32),
        pltpu.VMEM((RPW, AP), jnp.float32),
        pltpu.SemaphoreType.DMA,
        pltpu.SemaphoreType.DMA,
    ],
)
def _rand_hash_proj(x_hbm, sel_hbm, bin_hbm, out_hbm,
                    xb0, xb1, sb0, sb1, bb0, bb1, acc, sem0, sem1):
    cid = lax.axis_index("c")
    sid = lax.axis_index("s")
    wid = sid * NC + cid
    row0 = wid * RPW

    lane = lax.iota(jnp.int32, L)
    lanef = lane
    zeros16 = jnp.zeros((L,), jnp.float32)

    bufs = ((xb0, sb0, bb0, sem0), (xb1, sb1, bb1, sem1))

    def copies(c, slot_bufs):
        xb, sb, bb, sem = slot_bufs
        col = c * CH
        return (
            pltpu.make_async_copy(
                x_hbm.at[pl.ds(row0, RPW), pl.ds(col, CH)],
                xb.at[:, pl.ds(0, CH)],
                sem,
            ),
            pltpu.make_async_copy(sel_hbm.at[pl.ds(col, CH)], sb, sem),
            pltpu.make_async_copy(bin_hbm.at[pl.ds(col, CH)], bb, sem),
        )

    def start(c, slot_bufs):
        for cp in copies(c, slot_bufs):
            cp.start()

    def wait(c, slot_bufs):
        for cp in copies(c, slot_bufs):
            cp.wait()

    def zero_row(b, _):
        def zero_grp(j, _):
            acc[b, pl.ds(j * L, L)] = zeros16
            return None
        lax.fori_loop(0, 2 * OUT // L, zero_grp, None)
        return None
    lax.fori_loop(0, RPW, zero_row, None)

    start(0, bufs[0])

    def do_chunk(c, slot_bufs, other_bufs):
        xb, sb, bb, _ = slot_bufs

        @pl.when(c + 1 < NCHUNK)
        def _():
            start(c + 1, other_bufs)

        wait(c, slot_bufs)

        def fuse(g, _):
            sl = pl.ds(g * L, L)
            sb[sl] = sb[sl] + bb[sl] * OUT
            return None
        lax.fori_loop(0, CH // L, fuse, None)

        def grp(g, _):
            binv = sb[pl.ds(g * L, L)]
            for l in range(L):
                bsel = binv.at[jnp.full((L,), l, jnp.int32)].get(
                    mode="promise_in_bounds")
                fvec = jnp.full((L,), g * L + l, jnp.int32)
                for h in range(2):
                    bl = lanef + (h * L)
                    xv = plsc.load_gather(xb, [bl, fvec])
                    plsc.addupdate_scatter(acc, [bl, bsel], xv)
            return None
        lax.fori_loop(0, CH // L, grp, None)

    def outer(i, _):
        do_chunk(2 * i, bufs[0], bufs[1])
        do_chunk(2 * i + 1, bufs[1], bufs[0])
        return None
    lax.fori_loop(0, NCHUNK // 2, outer, None)

    def diff_row(b, _):
        def diff_grp(j, _):
            sl = pl.ds(j * L, L)
            sh = pl.ds(OUT + j * L, L)
            acc[b, sl] = acc[b, sh] - acc[b, sl]
            return None
        lax.fori_loop(0, OUT // L, diff_grp, None)
        return None
    lax.fori_loop(0, RPW, diff_row, None)

    pltpu.sync_copy(acc.at[:, pl.ds(0, OUT)],
                    out_hbm.at[pl.ds(row0, RPW), :])


def kernel(x, selector, binary):
    return _rand_hash_proj(x, selector, binary)


# speedup vs baseline: 1.7586x; 1.7586x over previous
import functools

import jax
import jax.numpy as jnp
from jax import lax
from jax.experimental import pallas as pl
from jax.experimental.pallas import tpu as pltpu
from jax.experimental.pallas import tpu_sc as plsc

OUT = 1024
NF = 65536
B = 1024

NC = 2
NS = 16
L = 16
NW = NC * NS
RPW = B // NW
CH = 512
NCHUNK = NF // CH
XP = CH + 1
AP = 2 * OUT + 1

_mesh = plsc.VectorSubcoreMesh(
    core_axis_name="c", subcore_axis_name="s", num_cores=NC, num_subcores=NS
)


@functools.partial(
    pl.kernel,
    out_type=jax.ShapeDtypeStruct((B, OUT), jnp.float32),
    mesh=_mesh,
    compiler_params=pltpu.CompilerParams(use_tc_tiling_on_sc=False,
                                          needs_layout_passes=False),
    scratch_types=[
        pltpu.VMEM((RPW, XP), jnp.float32),
        pltpu.VMEM((RPW, XP), jnp.float32),
        pltpu.VMEM((CH,), jnp.int32),
        pltpu.VMEM((CH,), jnp.int32),
        pltpu.VMEM((CH,), jnp.int32),
        pltpu.VMEM((CH,), jnp.int32),
        pltpu.VMEM((RPW, AP), jnp.float32),
        pltpu.SemaphoreType.DMA,
        pltpu.SemaphoreType.DMA,
    ],
)
def _rand_hash_proj(x_hbm, sel_hbm, bin_hbm, out_hbm,
                    xb0, xb1, sb0, sb1, bb0, bb1, acc, sem0, sem1):
    cid = lax.axis_index("c")
    sid = lax.axis_index("s")
    wid = sid * NC + cid
    row0 = wid * RPW

    lane = lax.iota(jnp.int32, L)
    lanef = lane
    zeros16 = jnp.zeros((L,), jnp.float32)

    bufs = ((xb0, sb0, bb0, sem0), (xb1, sb1, bb1, sem1))

    def copies(c, slot_bufs):
        xb, sb, bb, sem = slot_bufs
        col = c * CH
        return (
            pltpu.make_async_copy(
                x_hbm.at[pl.ds(row0, RPW), pl.ds(col, CH)],
                xb.at[:, pl.ds(0, CH)],
                sem,
            ),
            pltpu.make_async_copy(sel_hbm.at[pl.ds(col, CH)], sb, sem),
            pltpu.make_async_copy(bin_hbm.at[pl.ds(col, CH)], bb, sem),
        )

    def start(c, slot_bufs):
        for cp in copies(c, slot_bufs):
            cp.start()

    def wait(c, slot_bufs):
        for cp in copies(c, slot_bufs):
            cp.wait()

    def zero_row(b, _):
        def zero_grp(j, _):
            acc[b, pl.ds(j * L, L)] = zeros16
            return None
        lax.fori_loop(0, 2 * OUT // L, zero_grp, None)
        return None
    lax.fori_loop(0, RPW, zero_row, None)

    start(0, bufs[0])

    def do_chunk(c, slot_bufs, other_bufs):
        xb, sb, bb, _ = slot_bufs

        @pl.when(c + 1 < NCHUNK)
        def _():
            start(c + 1, other_bufs)

        wait(c, slot_bufs)

        def fuse(g, _):
            sl = pl.ds(g * L, L)
            sb[sl] = sb[sl] + bb[sl] * OUT
            return None
        lax.fori_loop(0, CH // L, fuse, None)

        PIPE = 6

        def grp(g, _):
            binv = sb[pl.ds(g * L, L)]
            pairs = []
            for l in range(L):
                bsel = binv.at[jnp.full((L,), l, jnp.int32)].get(
                    mode="promise_in_bounds")
                fvec = jnp.full((L,), g * L + l, jnp.int32)
                for h in range(2):
                    bl = lanef + (h * L)
                    pairs.append((bl, fvec, bsel))
            inflight = [plsc.load_gather(xb, [bl, fv]) for bl, fv, _ in pairs[:PIPE]]
            for k, (bl, fv, bs) in enumerate(pairs):
                if k + PIPE < len(pairs):
                    nbl, nfv, _ = pairs[k + PIPE]
                    inflight.append(plsc.load_gather(xb, [nbl, nfv]))
                plsc.addupdate_scatter(acc, [bl, bs], inflight[k])
            return None
        lax.fori_loop(0, CH // L, grp, None)

    def outer(i, _):
        do_chunk(2 * i, bufs[0], bufs[1])
        do_chunk(2 * i + 1, bufs[1], bufs[0])
        return None
    lax.fori_loop(0, NCHUNK // 2, outer, None)

    def diff_row(b, _):
        def diff_grp(j, _):
            sl = pl.ds(j * L, L)
            sh = pl.ds(OUT + j * L, L)
            acc[b, sl] = acc[b, sh] - acc[b, sl]
            return None
        lax.fori_loop(0, OUT // L, diff_grp, None)
        return None
    lax.fori_loop(0, RPW, diff_row, None)

    pltpu.sync_copy(acc.at[:, pl.ds(0, OUT)],
                    out_hbm.at[pl.ds(row0, RPW), :])


def kernel(x, selector, binary):
    return _rand_hash_proj(x, selector, binary)
